# baseline (speedup 1.0000x reference)
import functools

import jax
import jax.numpy as jnp
from jax import lax
from jax.experimental import pallas as pl
from jax.experimental.pallas import tpu as pltpu
from jax.experimental.pallas import tpu_sc as plsc

N = 10000
E = 320000
D_IN = 128
D_HID = 64

NC, NS = 2, 16
NW = NC * NS
CHUNK = 128
CH_PER_W = 80
E_PAD = CH_PER_W * NW * CHUNK
NBUF = 4
TRASH = N
ROWS_A = 640
ROWS_LAST = N - (NS - 1) * ROWS_A

_mesh = plsc.VectorSubcoreMesh(core_axis_name="c", subcore_axis_name="s")


def _per_tile_rows(sid, copy):
    @pl.when(sid < NS - 1)
    def _():
        copy(sid * ROWS_A, ROWS_A)

    @pl.when(sid == NS - 1)
    def _():
        copy((NS - 1) * ROWS_A, ROWS_LAST)


DEG_W = 16


@functools.partial(
    pl.kernel,
    out_type=jax.ShapeDtypeStruct((NC, N, DEG_W), jnp.float32),
    mesh=_mesh,
    scratch_types=[
        pltpu.VMEM((CH_PER_W, CHUNK), jnp.int32),
        pltpu.VMEM((CHUNK, DEG_W), jnp.float32),
        pltpu.VMEM_SHARED((N + 8, DEG_W), jnp.float32),
        pltpu.SemaphoreType.DMA,
        pltpu.SemaphoreType.DMA,
    ],
    compiler_params=pltpu.CompilerParams(use_tc_tiling_on_sc=False),
)
def _sc_degree(dst_hbm, zcol_hbm, ones_hbm, deg_hbm, dst_v, ones_v, acc_s,
               sem_i, sem_s):
    cid = lax.axis_index("c")
    sid = lax.axis_index("s")
    wid = sid * NC + cid
    cpd = pltpu.async_copy(dst_hbm.at[wid], dst_v, sem_i)

    def init(off, cnt):
        pltpu.sync_copy(zcol_hbm.at[pl.ds(0, cnt)], acc_s.at[pl.ds(off, cnt)])

    _per_tile_rows(sid, init)
    pltpu.sync_copy(ones_hbm, ones_v)
    cpd.wait()
    plsc.subcore_barrier()

    def body(g, carry):
        for j in range(NBUF):
            c = NBUF * g + j
            pltpu.async_copy(ones_v, acc_s.at[dst_v.at[c]], sem_s, add=True)
        for j in range(NBUF):
            c = NBUF * g + j
            pltpu.make_async_copy(ones_v, acc_s.at[dst_v.at[c]], sem_s).wait()
        return carry

    lax.fori_loop(0, CH_PER_W // NBUF, body, 0)
    plsc.subcore_barrier()

    def writeout(off, cnt):
        pltpu.sync_copy(acc_s.at[pl.ds(off, cnt)],
                        deg_hbm.at[cid, pl.ds(off, cnt)])

    _per_tile_rows(sid, writeout)


@functools.partial(
    pl.kernel,
    out_type=jax.ShapeDtypeStruct((NC, N, D_HID), jnp.float32),
    mesh=_mesh,
    scratch_types=[
        pltpu.VMEM((CH_PER_W, CHUNK), jnp.int32),
        pltpu.VMEM((CH_PER_W, CHUNK), jnp.int32),
        pltpu.VMEM((CHUNK, D_HID), jnp.float32),
        pltpu.VMEM((CHUNK, D_HID), jnp.float32),
        pltpu.VMEM((CHUNK, D_HID), jnp.float32),
        pltpu.VMEM((CHUNK, D_HID), jnp.float32),
        pltpu.VMEM_SHARED((N + 8, D_HID), jnp.float32),
        pltpu.SemaphoreType.DMA,
        pltpu.SemaphoreType.DMA,
        pltpu.SemaphoreType.DMA,
        pltpu.SemaphoreType.DMA,
        pltpu.SemaphoreType.DMA,
    ],
    compiler_params=pltpu.CompilerParams(use_tc_tiling_on_sc=False),
)
def _sc_aggregate(table_hbm, src_hbm, dst_hbm, zeros_hbm, out_hbm,
                  src_v, dst_v, b0, b1, b2, b3, acc_s,
                  sem_i, s0, s1, s2, s3):
    cid = lax.axis_index("c")
    sid = lax.axis_index("s")
    wid = sid * NC + cid
    bufs = (b0, b1, b2, b3)
    sems = (s0, s1, s2, s3)

    cps = pltpu.async_copy(src_hbm.at[wid], src_v, sem_i)
    cpd = pltpu.async_copy(dst_hbm.at[wid], dst_v, sem_i)

    def init(off, cnt):
        @pl.when(cid == 0)
        def _():
            pltpu.sync_copy(table_hbm.at[pl.ds(off, cnt)],
                            acc_s.at[pl.ds(off, cnt)])

        @pl.when(cid == 1)
        def _():
            pltpu.sync_copy(zeros_hbm.at[pl.ds(0, cnt)],
                            acc_s.at[pl.ds(off, cnt)])

    _per_tile_rows(sid, init)
    cps.wait()
    cpd.wait()
    for j in range(NBUF):
        pltpu.async_copy(table_hbm.at[src_v.at[j]], bufs[j], sems[j])
    plsc.subcore_barrier()

    def body(g, carry):
        for j in range(NBUF):
            c = NBUF * g + j
            pltpu.make_async_copy(table_hbm.at[src_v.at[c]],
                                  bufs[j], sems[j]).wait()
            pltpu.sync_copy(bufs[j], acc_s.at[dst_v.at[c]], add=True)

            @pl.when(c + NBUF < CH_PER_W)
            def _():
                pltpu.async_copy(table_hbm.at[src_v.at[c + NBUF]],
                                 bufs[j], sems[j])
        return carry

    lax.fori_loop(0, CH_PER_W // NBUF, body, 0)
    plsc.subcore_barrier()

    def writeout(off, cnt):
        pltpu.sync_copy(acc_s.at[pl.ds(off, cnt)],
                        out_hbm.at[cid, pl.ds(off, cnt)])

    _per_tile_rows(sid, writeout)


BLK = 1000


def _tc_mm_scale_body(x_ref, w_ref, d0_ref, d1_ref, o_ref):
    dinv = lax.rsqrt(d0_ref[...] + d1_ref[...] + 1.0)
    o_ref[...] = jnp.dot(x_ref[...], w_ref[...],
                         preferred_element_type=jnp.float32) * dinv


def _tc_mm_scale(x, w, d0, d1):
    m, k = x.shape
    n = w.shape[1]
    return pl.pallas_call(
        _tc_mm_scale_body,
        grid=(m // BLK,),
        in_specs=[
            pl.BlockSpec((BLK, k), lambda i: (i, 0)),
            pl.BlockSpec((k, n), lambda i: (0, 0)),
            pl.BlockSpec((BLK, 1), lambda i: (i, 0)),
            pl.BlockSpec((BLK, 1), lambda i: (i, 0)),
        ],
        out_specs=pl.BlockSpec((BLK, n), lambda i: (i, 0)),
        out_shape=jax.ShapeDtypeStruct((m, n), jnp.float32),
    )(x, w, d0, d1)


def _tc_layer2_body(a0_ref, a1_ref, d0_ref, d1_ref, b_ref, w_ref, o_ref):
    dinv = lax.rsqrt(d0_ref[...] + d1_ref[...] + 1.0)
    h = jnp.maximum((a0_ref[...] + a1_ref[...]) * dinv + b_ref[...], 0.0)
    o_ref[...] = jnp.dot(h, w_ref[...],
                         preferred_element_type=jnp.float32) * dinv


def _tc_layer2(a0, a1, d0, d1, b, w):
    m, n = a0.shape
    return pl.pallas_call(
        _tc_layer2_body,
        grid=(m // BLK,),
        in_specs=[
            pl.BlockSpec((BLK, n), lambda i: (i, 0)),
            pl.BlockSpec((BLK, n), lambda i: (i, 0)),
            pl.BlockSpec((BLK, 1), lambda i: (i, 0)),
            pl.BlockSpec((BLK, 1), lambda i: (i, 0)),
            pl.BlockSpec((1, n), lambda i: (0, 0)),
            pl.BlockSpec((n, n), lambda i: (0, 0)),
        ],
        out_specs=pl.BlockSpec((BLK, n), lambda i: (i, 0)),
        out_shape=jax.ShapeDtypeStruct((m, n), jnp.float32),
    )(a0, a1, d0, d1, b, w)


def _tc_final_body(a0_ref, a1_ref, d0_ref, d1_ref, b_ref, w_ref, o_ref):
    @pl.when(pl.program_id(0) == 0)
    def _():
        o_ref[...] = jnp.zeros_like(o_ref)

    dinv = lax.rsqrt(d0_ref[...] + d1_ref[...] + 1.0)
    h = jnp.maximum((a0_ref[...] + a1_ref[...]) * dinv + b_ref[...], 0.0)
    o_ref[...] += jnp.sum(
        jnp.dot(h, w_ref[...], preferred_element_type=jnp.float32),
        axis=0, keepdims=True)


def _tc_final(a0, a1, d0, d1, b, w):
    m, n = a0.shape
    return pl.pallas_call(
        _tc_final_body,
        grid=(m // BLK,),
        in_specs=[
            pl.BlockSpec((BLK, n), lambda i: (i, 0)),
            pl.BlockSpec((BLK, n), lambda i: (i, 0)),
            pl.BlockSpec((BLK, 1), lambda i: (i, 0)),
            pl.BlockSpec((BLK, 1), lambda i: (i, 0)),
            pl.BlockSpec((1, n), lambda i: (0, 0)),
            pl.BlockSpec((n, 1), lambda i: (0, 0)),
        ],
        out_specs=pl.BlockSpec((1, 1), lambda i: (0, 0)),
        out_shape=jax.ShapeDtypeStruct((1, 1), jnp.float32),
    )(a0, a1, d0, d1, b, w)


def kernel(x, edge_index, W1, b1, W2, b2, Wfc, bfc):
    src = edge_index[0].astype(jnp.int32)
    dst = edge_index[1].astype(jnp.int32)
    pad = E_PAD - E
    srcp = jnp.concatenate([src, jnp.zeros((pad,), jnp.int32)])
    srcp = srcp.reshape(NW, CH_PER_W, CHUNK)
    dstp = jnp.concatenate([dst, jnp.full((pad,), TRASH, jnp.int32)])
    dstp = dstp.reshape(NW, CH_PER_W, CHUNK)
    zeros64 = jnp.zeros((ROWS_A, D_HID), jnp.float32)
    zcol = jnp.zeros((ROWS_A, DEG_W), jnp.float32)
    ones = jnp.ones((CHUNK, DEG_W), jnp.float32)

    deg = _sc_degree(dstp, zcol, ones)
    d0, d1 = deg[0, :, 0:1], deg[1, :, 0:1]

    h1p = _tc_mm_scale(x, W1, d0, d1)
    acc1 = _sc_aggregate(h1p, srcp, dstp, zeros64)
    h2p = _tc_layer2(acc1[0], acc1[1], d0, d1, b1.reshape(1, -1), W2)
    acc2 = _sc_aggregate(h2p, srcp, dstp, zeros64)
    s = _tc_final(acc2[0], acc2[1], d0, d1, b2.reshape(1, -1), Wfc)

    return s.reshape((1,)) / N + bfc


# speedup vs baseline: 1.2705x; 1.2705x over previous
import functools

import jax
import jax.numpy as jnp
from jax import lax
from jax.experimental import pallas as pl
from jax.experimental.pallas import tpu as pltpu
from jax.experimental.pallas import tpu_sc as plsc

N = 10000
E = 320000
D_IN = 128
D_HID = 64

NC, NS = 2, 16
NW = NC * NS
CHUNK = 128
CH_PER_W = 80
E_PAD = CH_PER_W * NW * CHUNK
NBUF = 4
TRASH_ROWS = 512
ROWS_A = 640
ROWS_LAST = N - (NS - 1) * ROWS_A

_mesh = plsc.VectorSubcoreMesh(core_axis_name="c", subcore_axis_name="s")


def _per_tile_rows(sid, copy):
    @pl.when(sid < NS - 1)
    def _():
        copy(sid * ROWS_A, ROWS_A)

    @pl.when(sid == NS - 1)
    def _():
        copy((NS - 1) * ROWS_A, ROWS_LAST)


DEG_W = 16


@functools.partial(
    pl.kernel,
    out_type=jax.ShapeDtypeStruct((NC, N, DEG_W), jnp.float32),
    mesh=_mesh,
    scratch_types=[
        pltpu.VMEM((CH_PER_W, CHUNK), jnp.int32),
        pltpu.VMEM((CHUNK, DEG_W), jnp.float32),
        pltpu.VMEM_SHARED((N + TRASH_ROWS, DEG_W), jnp.float32),
        pltpu.SemaphoreType.DMA,
        pltpu.SemaphoreType.DMA,
    ],
    compiler_params=pltpu.CompilerParams(use_tc_tiling_on_sc=False),
)
def _sc_degree(dst_hbm, zcol_hbm, ones_hbm, deg_hbm, dst_v, ones_v, acc_s,
               sem_i, sem_s):
    cid = lax.axis_index("c")
    sid = lax.axis_index("s")
    wid = sid * NC + cid
    cpd = pltpu.async_copy(dst_hbm.at[wid], dst_v, sem_i)

    def init(off, cnt):
        pltpu.sync_copy(zcol_hbm.at[pl.ds(0, cnt)], acc_s.at[pl.ds(off, cnt)])

    _per_tile_rows(sid, init)
    pltpu.sync_copy(ones_hbm, ones_v)
    cpd.wait()
    plsc.subcore_barrier()

    def body(g, carry):
        for j in range(NBUF):
            c = NBUF * g + j
            pltpu.async_copy(ones_v, acc_s.at[dst_v.at[c]], sem_s, add=True)
        for j in range(NBUF):
            c = NBUF * g + j
            pltpu.make_async_copy(ones_v, acc_s.at[dst_v.at[c]], sem_s).wait()
        return carry

    lax.fori_loop(0, CH_PER_W // NBUF, body, 0)
    plsc.subcore_barrier()

    def writeout(off, cnt):
        pltpu.sync_copy(acc_s.at[pl.ds(off, cnt)],
                        deg_hbm.at[cid, pl.ds(off, cnt)])

    _per_tile_rows(sid, writeout)


@functools.partial(
    pl.kernel,
    out_type=jax.ShapeDtypeStruct((NC, N, D_HID), jnp.float32),
    mesh=_mesh,
    scratch_types=[
        pltpu.VMEM((CH_PER_W, CHUNK), jnp.int32),
        pltpu.VMEM((CH_PER_W, CHUNK), jnp.int32),
        pltpu.VMEM((CHUNK, D_HID), jnp.float32),
        pltpu.VMEM((CHUNK, D_HID), jnp.float32),
        pltpu.VMEM((CHUNK, D_HID), jnp.float32),
        pltpu.VMEM((CHUNK, D_HID), jnp.float32),
        pltpu.VMEM_SHARED((N + TRASH_ROWS, D_HID), jnp.float32),
        pltpu.SemaphoreType.DMA,
        pltpu.SemaphoreType.DMA,
        pltpu.SemaphoreType.DMA,
        pltpu.SemaphoreType.DMA,
        pltpu.SemaphoreType.DMA,
    ],
    compiler_params=pltpu.CompilerParams(use_tc_tiling_on_sc=False),
)
def _sc_aggregate(table_hbm, src_hbm, dst_hbm, zeros_hbm, out_hbm,
                  src_v, dst_v, b0, b1, b2, b3, acc_s,
                  sem_i, s0, s1, s2, s3):
    cid = lax.axis_index("c")
    sid = lax.axis_index("s")
    wid = sid * NC + cid
    bufs = (b0, b1, b2, b3)
    sems = (s0, s1, s2, s3)

    cps = pltpu.async_copy(src_hbm.at[wid], src_v, sem_i)
    cpd = pltpu.async_copy(dst_hbm.at[wid], dst_v, sem_i)

    def init(off, cnt):
        @pl.when(cid == 0)
        def _():
            pltpu.sync_copy(table_hbm.at[pl.ds(off, cnt)],
                            acc_s.at[pl.ds(off, cnt)])

        @pl.when(cid == 1)
        def _():
            pltpu.sync_copy(zeros_hbm.at[pl.ds(0, cnt)],
                            acc_s.at[pl.ds(off, cnt)])

    _per_tile_rows(sid, init)
    cps.wait()
    cpd.wait()
    for j in range(NBUF):
        pltpu.async_copy(table_hbm.at[src_v.at[j]], bufs[j], sems[j])
    plsc.subcore_barrier()

    def body(g, carry):
        for j in range(NBUF):
            c = NBUF * g + j
            pltpu.make_async_copy(table_hbm.at[src_v.at[c]],
                                  bufs[j], sems[j]).wait()
            pltpu.sync_copy(bufs[j], acc_s.at[dst_v.at[c]], add=True)

            @pl.when(c + NBUF < CH_PER_W)
            def _():
                pltpu.async_copy(table_hbm.at[src_v.at[c + NBUF]],
                                 bufs[j], sems[j])
        return carry

    lax.fori_loop(0, CH_PER_W // NBUF, body, 0)
    plsc.subcore_barrier()

    def writeout(off, cnt):
        pltpu.sync_copy(acc_s.at[pl.ds(off, cnt)],
                        out_hbm.at[cid, pl.ds(off, cnt)])

    _per_tile_rows(sid, writeout)


BLK = 1000


def _tc_mm_body(x_ref, w_ref, o_ref):
    o_ref[...] = jnp.dot(x_ref[...], w_ref[...],
                         preferred_element_type=jnp.float32)


def _tc_mm(x, w):
    m, k = x.shape
    n = w.shape[1]
    return pl.pallas_call(
        _tc_mm_body,
        grid=(m // BLK,),
        in_specs=[
            pl.BlockSpec((BLK, k), lambda i: (i, 0)),
            pl.BlockSpec((k, n), lambda i: (0, 0)),
        ],
        out_specs=pl.BlockSpec((BLK, n), lambda i: (i, 0)),
        out_shape=jax.ShapeDtypeStruct((m, n), jnp.float32),
    )(x, w)


def _tc_scale_body(h_ref, d0_ref, d1_ref, o_ref):
    dinv = 1.0 / jnp.sqrt(d0_ref[...] + d1_ref[...] + 1.0)
    o_ref[...] = h_ref[...] * dinv


def _tc_scale(h, d0, d1):
    m, n = h.shape
    return pl.pallas_call(
        _tc_scale_body,
        grid=(m // BLK,),
        in_specs=[
            pl.BlockSpec((BLK, n), lambda i: (i, 0)),
            pl.BlockSpec((BLK, 1), lambda i: (i, 0)),
            pl.BlockSpec((BLK, 1), lambda i: (i, 0)),
        ],
        out_specs=pl.BlockSpec((BLK, n), lambda i: (i, 0)),
        out_shape=jax.ShapeDtypeStruct((m, n), jnp.float32),
    )(h, d0, d1)


def _tc_layer2_body(a0_ref, a1_ref, d0_ref, d1_ref, b_ref, w_ref, o_ref):
    dinv = 1.0 / jnp.sqrt(d0_ref[...] + d1_ref[...] + 1.0)
    h = jnp.maximum((a0_ref[...] + a1_ref[...]) * dinv + b_ref[...], 0.0)
    o_ref[...] = jnp.dot(h, w_ref[...],
                         preferred_element_type=jnp.float32) * dinv


def _tc_layer2(a0, a1, d0, d1, b, w):
    m, n = a0.shape
    return pl.pallas_call(
        _tc_layer2_body,
        grid=(m // BLK,),
        in_specs=[
            pl.BlockSpec((BLK, n), lambda i: (i, 0)),
            pl.BlockSpec((BLK, n), lambda i: (i, 0)),
            pl.BlockSpec((BLK, 1), lambda i: (i, 0)),
            pl.BlockSpec((BLK, 1), lambda i: (i, 0)),
            pl.BlockSpec((1, n), lambda i: (0, 0)),
            pl.BlockSpec((n, n), lambda i: (0, 0)),
        ],
        out_specs=pl.BlockSpec((BLK, n), lambda i: (i, 0)),
        out_shape=jax.ShapeDtypeStruct((m, n), jnp.float32),
    )(a0, a1, d0, d1, b, w)


def _tc_final_body(a0_ref, a1_ref, d0_ref, d1_ref, b_ref, w_ref, o_ref):
    @pl.when(pl.program_id(0) == 0)
    def _():
        o_ref[...] = jnp.zeros_like(o_ref)

    dinv = 1.0 / jnp.sqrt(d0_ref[...] + d1_ref[...] + 1.0)
    h = jnp.maximum((a0_ref[...] + a1_ref[...]) * dinv + b_ref[...], 0.0)
    o_ref[...] += jnp.sum(
        jnp.dot(h, w_ref[...], preferred_element_type=jnp.float32),
        axis=0, keepdims=True)


def _tc_final(a0, a1, d0, d1, b, w):
    m, n = a0.shape
    return pl.pallas_call(
        _tc_final_body,
        grid=(m // BLK,),
        in_specs=[
            pl.BlockSpec((BLK, n), lambda i: (i, 0)),
            pl.BlockSpec((BLK, n), lambda i: (i, 0)),
            pl.BlockSpec((BLK, 1), lambda i: (i, 0)),
            pl.BlockSpec((BLK, 1), lambda i: (i, 0)),
            pl.BlockSpec((1, n), lambda i: (0, 0)),
            pl.BlockSpec((n, 1), lambda i: (0, 0)),
        ],
        out_specs=pl.BlockSpec((1, 1), lambda i: (0, 0)),
        out_shape=jax.ShapeDtypeStruct((1, 1), jnp.float32),
    )(a0, a1, d0, d1, b, w)


def kernel(x, edge_index, W1, b1, W2, b2, Wfc, bfc):
    src = edge_index[0].astype(jnp.int32)
    dst = edge_index[1].astype(jnp.int32)
    pad = E_PAD - E
    srcp = jnp.concatenate([src, jnp.zeros((pad,), jnp.int32)])
    srcp = srcp.reshape(NW, CH_PER_W, CHUNK)
    trash_idx = N + (jnp.arange(pad, dtype=jnp.int32) % TRASH_ROWS)
    dstp = jnp.concatenate([dst, trash_idx])
    dstp = dstp.reshape(NW, CH_PER_W, CHUNK)
    zeros64 = jnp.zeros((ROWS_A, D_HID), jnp.float32)
    zcol = jnp.zeros((ROWS_A, DEG_W), jnp.float32)
    ones = jnp.ones((CHUNK, DEG_W), jnp.float32)

    h1 = _tc_mm(x, W1)
    deg = _sc_degree(dstp, zcol, ones)
    d0, d1 = deg[0, :, 0:1], deg[1, :, 0:1]

    h1p = _tc_scale(h1, d0, d1)
    acc1 = _sc_aggregate(h1p, srcp, dstp, zeros64)
    h2p = _tc_layer2(acc1[0], acc1[1], d0, d1, b1.reshape(1, -1), W2)
    acc2 = _sc_aggregate(h2p, srcp, dstp, zeros64)
    s = _tc_final(acc2[0], acc2[1], d0, d1, b2.reshape(1, -1), Wfc)

    return s.reshape((1,)) / N + bfc


# speedup vs baseline: 2.8991x; 2.2818x over previous
import functools

import jax
import jax.numpy as jnp
from jax import lax
from jax.experimental import pallas as pl
from jax.experimental.pallas import tpu as pltpu
from jax.experimental.pallas import tpu_sc as plsc

N = 10000
E = 320000
D_IN = 128
D_HID = 64

NC, NS = 2, 16
NW = NC * NS
CHUNK = 128
CH_PER_W = 80
E_PAD = CH_PER_W * NW * CHUNK
NBUF = 4
TRASH_ROWS = 512
ROWS_A = 640
ROWS_LAST = N - (NS - 1) * ROWS_A

_mesh = plsc.VectorSubcoreMesh(core_axis_name="c", subcore_axis_name="s")


def _per_tile_rows(sid, copy):
    @pl.when(sid < NS - 1)
    def _():
        copy(sid * ROWS_A, ROWS_A)

    @pl.when(sid == NS - 1)
    def _():
        copy((NS - 1) * ROWS_A, ROWS_LAST)


DEG_W = 16


@functools.partial(
    pl.kernel,
    out_type=jax.ShapeDtypeStruct((NC, N, DEG_W), jnp.float32),
    mesh=_mesh,
    scratch_types=[
        pltpu.VMEM((CH_PER_W, CHUNK), jnp.int32),
        pltpu.VMEM((CHUNK, DEG_W), jnp.float32),
        pltpu.VMEM_SHARED((N + TRASH_ROWS, DEG_W), jnp.float32),
        pltpu.SemaphoreType.DMA,
        pltpu.SemaphoreType.DMA,
    ],
    compiler_params=pltpu.CompilerParams(use_tc_tiling_on_sc=False),
)
def _sc_degree(dst_hbm, zcol_hbm, ones_hbm, deg_hbm, dst_v, ones_v, acc_s,
               sem_i, sem_s):
    cid = lax.axis_index("c")
    sid = lax.axis_index("s")
    wid = sid * NC + cid
    cpd = pltpu.async_copy(dst_hbm.at[wid], dst_v, sem_i)

    def init(off, cnt):
        pltpu.sync_copy(zcol_hbm.at[pl.ds(0, cnt)], acc_s.at[pl.ds(off, cnt)])

    _per_tile_rows(sid, init)
    pltpu.sync_copy(ones_hbm, ones_v)
    cpd.wait()
    plsc.subcore_barrier()

    def body(g, carry):
        for j in range(NBUF):
            c = NBUF * g + j
            pltpu.async_copy(ones_v, acc_s.at[dst_v.at[c]], sem_s, add=True)
        for j in range(NBUF):
            c = NBUF * g + j
            pltpu.make_async_copy(ones_v, acc_s.at[dst_v.at[c]], sem_s).wait()
        return carry

    lax.fori_loop(0, CH_PER_W // NBUF, body, 0)
    plsc.subcore_barrier()

    def writeout(off, cnt):
        pltpu.sync_copy(acc_s.at[pl.ds(off, cnt)],
                        deg_hbm.at[cid, pl.ds(off, cnt)])

    _per_tile_rows(sid, writeout)


@functools.partial(
    pl.kernel,
    out_type=jax.ShapeDtypeStruct((NC, N, D_HID), jnp.float32),
    mesh=_mesh,
    scratch_types=[
        pltpu.VMEM((CH_PER_W, CHUNK), jnp.int32),
        pltpu.VMEM((CH_PER_W, CHUNK), jnp.int32),
        pltpu.VMEM((CHUNK, D_HID), jnp.float32),
        pltpu.VMEM((CHUNK, D_HID), jnp.float32),
        pltpu.VMEM((CHUNK, D_HID), jnp.float32),
        pltpu.VMEM((CHUNK, D_HID), jnp.float32),
        pltpu.VMEM_SHARED((N + TRASH_ROWS, D_HID), jnp.float32),
        pltpu.SemaphoreType.DMA,
        pltpu.SemaphoreType.DMA,
        pltpu.SemaphoreType.DMA,
        pltpu.SemaphoreType.DMA,
        pltpu.SemaphoreType.DMA,
    ],
    compiler_params=pltpu.CompilerParams(use_tc_tiling_on_sc=False),
)
def _sc_aggregate(table_hbm, src_hbm, dst_hbm, zeros_hbm, out_hbm,
                  src_v, dst_v, b0, b1, b2, b3, acc_s,
                  sem_i, s0, s1, s2, s3):
    cid = lax.axis_index("c")
    sid = lax.axis_index("s")
    wid = sid * NC + cid
    bufs = (b0, b1, b2, b3)
    sems = (s0, s1, s2, s3)

    cps = pltpu.async_copy(src_hbm.at[wid], src_v, sem_i)
    cpd = pltpu.async_copy(dst_hbm.at[wid], dst_v, sem_i)

    def init(off, cnt):
        @pl.when(cid == 0)
        def _():
            pltpu.sync_copy(table_hbm.at[pl.ds(off, cnt)],
                            acc_s.at[pl.ds(off, cnt)])

        @pl.when(cid == 1)
        def _():
            pltpu.sync_copy(zeros_hbm.at[pl.ds(0, cnt)],
                            acc_s.at[pl.ds(off, cnt)])

    _per_tile_rows(sid, init)
    cps.wait()
    cpd.wait()
    for j in range(NBUF):
        pltpu.async_copy(table_hbm.at[src_v.at[j]], bufs[j], sems[j])
    plsc.subcore_barrier()

    def body(g, carry):
        for j in range(NBUF):
            c = NBUF * g + j
            pltpu.make_async_copy(table_hbm.at[src_v.at[c]],
                                  bufs[j], sems[j]).wait()
            pltpu.sync_copy(bufs[j], acc_s.at[dst_v.at[c]], add=True)

            @pl.when(c + NBUF < CH_PER_W)
            def _():
                pltpu.async_copy(table_hbm.at[src_v.at[c + NBUF]],
                                 bufs[j], sems[j])
        return carry

    lax.fori_loop(0, CH_PER_W // NBUF, body, 0)
    plsc.subcore_barrier()

    def writeout(off, cnt):
        pltpu.sync_copy(acc_s.at[pl.ds(off, cnt)],
                        out_hbm.at[cid, pl.ds(off, cnt)])

    _per_tile_rows(sid, writeout)


BLK = 1000


def _tc_mm_body(x_ref, w_ref, o_ref):
    o_ref[...] = jnp.dot(x_ref[...], w_ref[...],
                         preferred_element_type=jnp.float32)


def _tc_mm(x, w):
    m, k = x.shape
    n = w.shape[1]
    return pl.pallas_call(
        _tc_mm_body,
        grid=(m // BLK,),
        in_specs=[
            pl.BlockSpec((BLK, k), lambda i: (i, 0)),
            pl.BlockSpec((k, n), lambda i: (0, 0)),
        ],
        out_specs=pl.BlockSpec((BLK, n), lambda i: (i, 0)),
        out_shape=jax.ShapeDtypeStruct((m, n), jnp.float32),
    )(x, w)


def _tc_scale_body(h_ref, d0_ref, d1_ref, o_ref):
    dinv = 1.0 / jnp.sqrt(d0_ref[...] + d1_ref[...] + 1.0)
    o_ref[...] = h_ref[...] * dinv


def _tc_scale(h, d0, d1):
    m, n = h.shape
    return pl.pallas_call(
        _tc_scale_body,
        grid=(m // BLK,),
        in_specs=[
            pl.BlockSpec((BLK, n), lambda i: (i, 0)),
            pl.BlockSpec((BLK, 1), lambda i: (i, 0)),
            pl.BlockSpec((BLK, 1), lambda i: (i, 0)),
        ],
        out_specs=pl.BlockSpec((BLK, n), lambda i: (i, 0)),
        out_shape=jax.ShapeDtypeStruct((m, n), jnp.float32),
    )(h, d0, d1)


def _tc_layer2_body(a0_ref, a1_ref, d0_ref, d1_ref, b_ref, w_ref, o_ref):
    dinv = 1.0 / jnp.sqrt(d0_ref[...] + d1_ref[...] + 1.0)
    h = jnp.maximum((a0_ref[...] + a1_ref[...]) * dinv + b_ref[...], 0.0)
    o_ref[...] = jnp.dot(h, w_ref[...],
                         preferred_element_type=jnp.float32) * dinv


def _tc_layer2(a0, a1, d0, d1, b, w):
    m, n = a0.shape
    return pl.pallas_call(
        _tc_layer2_body,
        grid=(m // BLK,),
        in_specs=[
            pl.BlockSpec((BLK, n), lambda i: (i, 0)),
            pl.BlockSpec((BLK, n), lambda i: (i, 0)),
            pl.BlockSpec((BLK, 1), lambda i: (i, 0)),
            pl.BlockSpec((BLK, 1), lambda i: (i, 0)),
            pl.BlockSpec((1, n), lambda i: (0, 0)),
            pl.BlockSpec((n, n), lambda i: (0, 0)),
        ],
        out_specs=pl.BlockSpec((BLK, n), lambda i: (i, 0)),
        out_shape=jax.ShapeDtypeStruct((m, n), jnp.float32),
    )(a0, a1, d0, d1, b, w)


def _tc_final_body(a0_ref, a1_ref, d0_ref, d1_ref, b_ref, w_ref, o_ref):
    @pl.when(pl.program_id(0) == 0)
    def _():
        o_ref[...] = jnp.zeros_like(o_ref)

    dinv = 1.0 / jnp.sqrt(d0_ref[...] + d1_ref[...] + 1.0)
    h = jnp.maximum((a0_ref[...] + a1_ref[...]) * dinv + b_ref[...], 0.0)
    o_ref[...] += jnp.sum(
        jnp.dot(h, w_ref[...], preferred_element_type=jnp.float32),
        axis=0, keepdims=True)


def _tc_final(a0, a1, d0, d1, b, w):
    m, n = a0.shape
    return pl.pallas_call(
        _tc_final_body,
        grid=(m // BLK,),
        in_specs=[
            pl.BlockSpec((BLK, n), lambda i: (i, 0)),
            pl.BlockSpec((BLK, n), lambda i: (i, 0)),
            pl.BlockSpec((BLK, 1), lambda i: (i, 0)),
            pl.BlockSpec((BLK, 1), lambda i: (i, 0)),
            pl.BlockSpec((1, n), lambda i: (0, 0)),
            pl.BlockSpec((n, 1), lambda i: (0, 0)),
        ],
        out_specs=pl.BlockSpec((1, 1), lambda i: (0, 0)),
        out_shape=jax.ShapeDtypeStruct((1, 1), jnp.float32),
    )(a0, a1, d0, d1, b, w)


def kernel(x, edge_index, W1, b1, W2, b2, Wfc, bfc):
    src = edge_index[0].astype(jnp.int32)
    dst = edge_index[1].astype(jnp.int32)
    pad = E_PAD - E
    pad_src = jnp.arange(pad, dtype=jnp.int32) % N
    srcp = jnp.concatenate([src, pad_src])
    srcp = srcp.reshape(NW, CH_PER_W, CHUNK)
    trash_idx = N + (jnp.arange(pad, dtype=jnp.int32) % TRASH_ROWS)
    dstp = jnp.concatenate([dst, trash_idx])
    dstp = dstp.reshape(NW, CH_PER_W, CHUNK)
    zeros64 = jnp.zeros((ROWS_A, D_HID), jnp.float32)
    zcol = jnp.zeros((ROWS_A, DEG_W), jnp.float32)
    ones = jnp.ones((CHUNK, DEG_W), jnp.float32)

    h1 = _tc_mm(x, W1)
    deg = _sc_degree(dstp, zcol, ones)
    d0, d1 = deg[0, :, 0:1], deg[1, :, 0:1]

    h1p = _tc_scale(h1, d0, d1)
    acc1 = _sc_aggregate(h1p, srcp, dstp, zeros64)
    h2p = _tc_layer2(acc1[0], acc1[1], d0, d1, b1.reshape(1, -1), W2)
    acc2 = _sc_aggregate(h2p, srcp, dstp, zeros64)
    s = _tc_final(acc2[0], acc2[1], d0, d1, b2.reshape(1, -1), Wfc)

    return s.reshape((1,)) / N + bfc


# speedup vs baseline: 3.2950x; 1.1366x over previous
import functools

import jax
import jax.numpy as jnp
from jax import lax
from jax.experimental import pallas as pl
from jax.experimental.pallas import tpu as pltpu
from jax.experimental.pallas import tpu_sc as plsc

N = 10000
E = 320000
D_IN = 128
D_HID = 64

NC, NS = 2, 16
NW = NC * NS
CHUNK = 128
CH_PER_W = 80
E_PAD = CH_PER_W * NW * CHUNK
NBUF = 4
TRASH_ROWS = 512
ROWS_A = 640
ROWS_LAST = N - (NS - 1) * ROWS_A

_mesh = plsc.VectorSubcoreMesh(core_axis_name="c", subcore_axis_name="s")


def _per_tile_rows(sid, copy):
    @pl.when(sid < NS - 1)
    def _():
        copy(sid * ROWS_A, ROWS_A)

    @pl.when(sid == NS - 1)
    def _():
        copy((NS - 1) * ROWS_A, ROWS_LAST)


DEG_W = 16


@functools.partial(
    pl.kernel,
    out_type=jax.ShapeDtypeStruct((NC, N, DEG_W), jnp.float32),
    mesh=_mesh,
    scratch_types=[
        pltpu.VMEM((CH_PER_W, CHUNK), jnp.int32),
        pltpu.VMEM((CHUNK, DEG_W), jnp.float32),
        pltpu.VMEM_SHARED((N + TRASH_ROWS, DEG_W), jnp.float32),
        pltpu.SemaphoreType.DMA,
        pltpu.SemaphoreType.DMA,
    ],
    compiler_params=pltpu.CompilerParams(use_tc_tiling_on_sc=False),
)
def _sc_degree(dst_hbm, zcol_hbm, ones_hbm, deg_hbm, dst_v, ones_v, acc_s,
               sem_i, sem_s):
    cid = lax.axis_index("c")
    sid = lax.axis_index("s")
    wid = sid * NC + cid
    cpd = pltpu.async_copy(dst_hbm.at[wid], dst_v, sem_i)

    def init(off, cnt):
        pltpu.sync_copy(zcol_hbm.at[pl.ds(0, cnt)], acc_s.at[pl.ds(off, cnt)])

    _per_tile_rows(sid, init)
    pltpu.sync_copy(ones_hbm, ones_v)
    cpd.wait()
    plsc.subcore_barrier()

    def body(g, carry):
        for j in range(NBUF):
            c = NBUF * g + j
            pltpu.async_copy(ones_v, acc_s.at[dst_v.at[c]], sem_s, add=True)
        for j in range(NBUF):
            c = NBUF * g + j
            pltpu.make_async_copy(ones_v, acc_s.at[dst_v.at[c]], sem_s).wait()
        return carry

    lax.fori_loop(0, CH_PER_W // NBUF, body, 0)
    plsc.subcore_barrier()

    def writeout(off, cnt):
        pltpu.sync_copy(acc_s.at[pl.ds(off, cnt)],
                        deg_hbm.at[cid, pl.ds(off, cnt)])

    _per_tile_rows(sid, writeout)


@functools.partial(
    pl.kernel,
    out_type=jax.ShapeDtypeStruct((NC, N, D_HID), jnp.float32),
    mesh=_mesh,
    scratch_types=[
        pltpu.VMEM((CH_PER_W, CHUNK), jnp.int32),
        pltpu.VMEM((CH_PER_W, CHUNK), jnp.int32),
        pltpu.VMEM((CHUNK, D_HID), jnp.float32),
        pltpu.VMEM((CHUNK, D_HID), jnp.float32),
        pltpu.VMEM((CHUNK, D_HID), jnp.float32),
        pltpu.VMEM((CHUNK, D_HID), jnp.float32),
        pltpu.VMEM_SHARED((N + TRASH_ROWS, D_HID), jnp.float32),
        pltpu.SemaphoreType.DMA,
        pltpu.SemaphoreType.DMA,
        pltpu.SemaphoreType.DMA,
        pltpu.SemaphoreType.DMA,
        pltpu.SemaphoreType.DMA,
    ],
    compiler_params=pltpu.CompilerParams(use_tc_tiling_on_sc=False),
)
def _sc_aggregate(table_hbm, src_hbm, dst_hbm, zeros_hbm, out_hbm,
                  src_v, dst_v, b0, b1, b2, b3, acc_s,
                  sem_i, s0, s1, s2, s3):
    cid = lax.axis_index("c")
    sid = lax.axis_index("s")
    wid = sid * NC + cid
    bufs = (b0, b1, b2, b3)
    sems = (s0, s1, s2, s3)

    cps = pltpu.async_copy(src_hbm.at[wid], src_v, sem_i)
    cpd = pltpu.async_copy(dst_hbm.at[wid], dst_v, sem_i)

    def init(off, cnt):
        @pl.when(cid == 0)
        def _():
            pltpu.sync_copy(table_hbm.at[pl.ds(off, cnt)],
                            acc_s.at[pl.ds(off, cnt)])

        @pl.when(cid == 1)
        def _():
            pltpu.sync_copy(zeros_hbm.at[pl.ds(0, cnt)],
                            acc_s.at[pl.ds(off, cnt)])

    _per_tile_rows(sid, init)
    cps.wait()
    cpd.wait()
    for j in range(NBUF):
        pltpu.async_copy(table_hbm.at[src_v.at[j]], bufs[j], sems[j])
    plsc.subcore_barrier()

    def body(g, carry):
        for j in range(NBUF):
            c = NBUF * g + j
            pltpu.make_async_copy(table_hbm.at[src_v.at[c]],
                                  bufs[j], sems[j]).wait()
            pltpu.sync_copy(bufs[j], acc_s.at[dst_v.at[c]], add=True)

            @pl.when(c + NBUF < CH_PER_W)
            def _():
                pltpu.async_copy(table_hbm.at[src_v.at[c + NBUF]],
                                 bufs[j], sems[j])
        return carry

    lax.fori_loop(0, CH_PER_W // NBUF, body, 0)
    plsc.subcore_barrier()

    def writeout(off, cnt):
        pltpu.sync_copy(acc_s.at[pl.ds(off, cnt)],
                        out_hbm.at[cid, pl.ds(off, cnt)])

    _per_tile_rows(sid, writeout)


BLK = 2000


def _tc_mm_body(x_ref, w_ref, o_ref):
    o_ref[...] = jnp.dot(x_ref[...], w_ref[...],
                         preferred_element_type=jnp.float32)


def _tc_mm(x, w):
    m, k = x.shape
    n = w.shape[1]
    return pl.pallas_call(
        _tc_mm_body,
        grid=(m // BLK,),
        in_specs=[
            pl.BlockSpec((BLK, k), lambda i: (i, 0)),
            pl.BlockSpec((k, n), lambda i: (0, 0)),
        ],
        out_specs=pl.BlockSpec((BLK, n), lambda i: (i, 0)),
        out_shape=jax.ShapeDtypeStruct((m, n), jnp.float32),
    )(x, w)


def _tc_scale_body(h_ref, d0_ref, d1_ref, o_ref):
    dinv = 1.0 / jnp.sqrt(d0_ref[0, :, 0:1] + d1_ref[0, :, 0:1] + 1.0)
    o_ref[...] = h_ref[...] * dinv


def _tc_scale(h, deg):
    m, n = h.shape
    return pl.pallas_call(
        _tc_scale_body,
        grid=(m // BLK,),
        in_specs=[
            pl.BlockSpec((BLK, n), lambda i: (i, 0)),
            pl.BlockSpec((1, BLK, DEG_W), lambda i: (0, i, 0)),
            pl.BlockSpec((1, BLK, DEG_W), lambda i: (1, i, 0)),
        ],
        out_specs=pl.BlockSpec((BLK, n), lambda i: (i, 0)),
        out_shape=jax.ShapeDtypeStruct((m, n), jnp.float32),
    )(h, deg, deg)


def _tc_layer2_body(a0_ref, a1_ref, d0_ref, d1_ref, b_ref, w_ref, o_ref):
    dinv = 1.0 / jnp.sqrt(d0_ref[0, :, 0:1] + d1_ref[0, :, 0:1] + 1.0)
    h = jnp.maximum((a0_ref[0] + a1_ref[0]) * dinv + b_ref[...], 0.0)
    o_ref[...] = jnp.dot(h, w_ref[...],
                         preferred_element_type=jnp.float32) * dinv


def _tc_layer2(acc, deg, b, w):
    n = acc.shape[2]
    m = acc.shape[1]
    return pl.pallas_call(
        _tc_layer2_body,
        grid=(m // BLK,),
        in_specs=[
            pl.BlockSpec((1, BLK, n), lambda i: (0, i, 0)),
            pl.BlockSpec((1, BLK, n), lambda i: (1, i, 0)),
            pl.BlockSpec((1, BLK, DEG_W), lambda i: (0, i, 0)),
            pl.BlockSpec((1, BLK, DEG_W), lambda i: (1, i, 0)),
            pl.BlockSpec((1, n), lambda i: (0, 0)),
            pl.BlockSpec((n, n), lambda i: (0, 0)),
        ],
        out_specs=pl.BlockSpec((BLK, n), lambda i: (i, 0)),
        out_shape=jax.ShapeDtypeStruct((m, n), jnp.float32),
    )(acc, acc, deg, deg, b, w)


def _tc_final_body(a0_ref, a1_ref, d0_ref, d1_ref, b_ref, w_ref, o_ref):
    @pl.when(pl.program_id(0) == 0)
    def _():
        o_ref[...] = jnp.zeros_like(o_ref)

    dinv = 1.0 / jnp.sqrt(d0_ref[0, :, 0:1] + d1_ref[0, :, 0:1] + 1.0)
    h = jnp.maximum((a0_ref[0] + a1_ref[0]) * dinv + b_ref[...], 0.0)
    o_ref[...] += jnp.sum(
        jnp.dot(h, w_ref[...], preferred_element_type=jnp.float32),
        axis=0, keepdims=True)


def _tc_final(acc, deg, b, w):
    n = acc.shape[2]
    m = acc.shape[1]
    return pl.pallas_call(
        _tc_final_body,
        grid=(m // BLK,),
        in_specs=[
            pl.BlockSpec((1, BLK, n), lambda i: (0, i, 0)),
            pl.BlockSpec((1, BLK, n), lambda i: (1, i, 0)),
            pl.BlockSpec((1, BLK, DEG_W), lambda i: (0, i, 0)),
            pl.BlockSpec((1, BLK, DEG_W), lambda i: (1, i, 0)),
            pl.BlockSpec((1, n), lambda i: (0, 0)),
            pl.BlockSpec((n, 1), lambda i: (0, 0)),
        ],
        out_specs=pl.BlockSpec((1, 1), lambda i: (0, 0)),
        out_shape=jax.ShapeDtypeStruct((1, 1), jnp.float32),
    )(acc, acc, deg, deg, b, w)


def kernel(x, edge_index, W1, b1, W2, b2, Wfc, bfc):
    src = edge_index[0].astype(jnp.int32)
    dst = edge_index[1].astype(jnp.int32)
    pad = E_PAD - E
    pad_src = jnp.arange(pad, dtype=jnp.int32) % N
    srcp = jnp.concatenate([src, pad_src])
    srcp = srcp.reshape(NW, CH_PER_W, CHUNK)
    trash_idx = N + (jnp.arange(pad, dtype=jnp.int32) % TRASH_ROWS)
    dstp = jnp.concatenate([dst, trash_idx])
    dstp = dstp.reshape(NW, CH_PER_W, CHUNK)
    zeros64 = jnp.zeros((ROWS_A, D_HID), jnp.float32)
    zcol = jnp.zeros((ROWS_A, DEG_W), jnp.float32)
    ones = jnp.ones((CHUNK, DEG_W), jnp.float32)

    h1 = _tc_mm(x, W1)
    deg = _sc_degree(dstp, zcol, ones)

    h1p = _tc_scale(h1, deg)
    acc1 = _sc_aggregate(h1p, srcp, dstp, zeros64)
    h2p = _tc_layer2(acc1, deg, b1.reshape(1, -1), W2)
    acc2 = _sc_aggregate(h2p, srcp, dstp, zeros64)
    s = _tc_final(acc2, deg, b2.reshape(1, -1), Wfc)

    return s.reshape((1,)) / N + bfc
